# baseline (speedup 1.0000x reference)
import functools

import jax
import jax.numpy as jnp
from jax import lax
from jax.experimental import pallas as pl
from jax.experimental.pallas import tpu as pltpu
from jax.experimental.pallas import tpu_sc as plsc

VOCAB = 100000
DIM = 128
BATCH = 1024
CTX = 50

NC = 2
NS = 16
L = 16
NW = NC * NS

NCHUNK = 2
CB = BATCH // NCHUNK


def _sc_gather_sum(x, emb_table):
    nb = x.shape[0]
    rows_per_w = nb // NW
    mesh = plsc.VectorSubcoreMesh(core_axis_name="c", subcore_axis_name="s")

    @functools.partial(
        pl.kernel,
        out_type=jax.ShapeDtypeStruct((nb, DIM), jnp.float32),
        mesh=mesh,
        scratch_types=[
            pltpu.VMEM((rows_per_w, CTX), jnp.int32),
            pltpu.VMEM((CTX, DIM), jnp.float32),
            pltpu.VMEM((rows_per_w, DIM), jnp.float32),
        ],
    )
    def k(x_hbm, tbl_hbm, out_hbm, idx_v, rows_v, acc_v):
        wid = lax.axis_index("s") * NC + lax.axis_index("c")
        base = wid * rows_per_w
        pltpu.sync_copy(x_hbm.at[pl.ds(base, rows_per_w)], idx_v)

        @pl.loop(0, rows_per_w)
        def _(r):
            pltpu.sync_copy(tbl_hbm.at[idx_v.at[r]], rows_v)
            for c in range(DIM // L):
                sl = pl.ds(c * L, L)
                s = rows_v[0, sl]
                for rr in range(1, CTX):
                    s = s + rows_v[rr, sl]
                acc_v[r, sl] = s

        pltpu.sync_copy(acc_v, out_hbm.at[pl.ds(base, rows_per_w)])

    return k(x, emb_table)


VB = 2048
_GRID = (VOCAB + VB - 1) // VB


def _tc_project_chunk(h, W, bcol, chunk, prev=None):
    if prev is None:

        def mm(h_ref, w_ref, b_ref, o_ref):
            hb = h_ref[...].astype(jnp.bfloat16)
            wb = w_ref[...].astype(jnp.bfloat16)
            acc = lax.dot_general(
                wb, hb, (((1,), (1,)), ((), ())),
                preferred_element_type=jnp.float32,
            )
            o_ref[...] = acc + b_ref[...]

        in_specs = [
            pl.BlockSpec((CB, DIM), lambda j: (0, 0)),
            pl.BlockSpec((VB, DIM), lambda j: (j, 0)),
            pl.BlockSpec((VB, 1), lambda j: (j, 0)),
        ]
        args = (h, W, bcol)
        aliases = {}
    else:

        def mm(h_ref, w_ref, b_ref, prev_ref, o_ref):
            hb = h_ref[...].astype(jnp.bfloat16)
            wb = w_ref[...].astype(jnp.bfloat16)
            acc = lax.dot_general(
                wb, hb, (((1,), (1,)), ((), ())),
                preferred_element_type=jnp.float32,
            )
            o_ref[...] = acc + b_ref[...]

        in_specs = [
            pl.BlockSpec((CB, DIM), lambda j: (0, 0)),
            pl.BlockSpec((VB, DIM), lambda j: (j, 0)),
            pl.BlockSpec((VB, 1), lambda j: (j, 0)),
            pl.BlockSpec(memory_space=pltpu.MemorySpace.HBM),
        ]
        args = (h, W, bcol, prev)
        aliases = {3: 0}

    return pl.pallas_call(
        mm,
        grid=(_GRID,),
        in_specs=in_specs,
        out_specs=pl.BlockSpec((VB, CB), lambda j, c=chunk: (j, c)),
        out_shape=jax.ShapeDtypeStruct((VOCAB, BATCH), jnp.float32),
        input_output_aliases=aliases,
        compiler_params=pltpu.CompilerParams(
            dimension_semantics=("arbitrary",),
        ),
    )(*args)


def kernel(x, emb_table, W, b):
    x = x.astype(jnp.int32)
    bcol = b.reshape(VOCAB, 1)
    hs = [
        _sc_gather_sum(x[k * CB:(k + 1) * CB], emb_table)
        for k in range(NCHUNK)
    ]
    lt = _tc_project_chunk(hs[0], W, bcol, 0)
    for k in range(1, NCHUNK):
        lt = _tc_project_chunk(hs[k], W, bcol, k, prev=lt)
    return lt.T


# speedup vs baseline: 1.3241x; 1.3241x over previous
import functools

import jax
import jax.numpy as jnp
from jax import lax
from jax.experimental import pallas as pl
from jax.experimental.pallas import tpu as pltpu
from jax.experimental.pallas import tpu_sc as plsc

VOCAB = 100000
DIM = 128
BATCH = 1024
CTX = 50

NC = 2
NS = 16
L = 16
NW = NC * NS
ROWS_PER_W = BATCH // NW


def _sc_gather_sum(x, emb_table):
    mesh = plsc.VectorSubcoreMesh(core_axis_name="c", subcore_axis_name="s")

    def _accum(rows_v, acc_v, r):
        for c in range(DIM // L):
            sl = pl.ds(c * L, L)
            s = rows_v[0, sl]
            for rr in range(1, CTX):
                s = s + rows_v[rr, sl]
            acc_v[r, sl] = s

    @functools.partial(
        pl.kernel,
        out_type=jax.ShapeDtypeStruct((BATCH, DIM), jnp.float32),
        mesh=mesh,
        scratch_types=[
            pltpu.VMEM((ROWS_PER_W, CTX), jnp.int32),
            pltpu.VMEM((CTX, DIM), jnp.float32),
            pltpu.VMEM((CTX, DIM), jnp.float32),
            pltpu.VMEM((ROWS_PER_W, DIM), jnp.float32),
            pltpu.SemaphoreType.DMA,
            pltpu.SemaphoreType.DMA,
        ],
    )
    def k(x_hbm, tbl_hbm, out_hbm, idx_v, rows_a, rows_b, acc_v, sem_a, sem_b):
        wid = lax.axis_index("s") * NC + lax.axis_index("c")
        base = wid * ROWS_PER_W
        pltpu.sync_copy(x_hbm.at[pl.ds(base, ROWS_PER_W)], idx_v)

        pltpu.async_copy(tbl_hbm.at[idx_v.at[0]], rows_a, sem_a)
        pltpu.async_copy(tbl_hbm.at[idx_v.at[1]], rows_b, sem_b)

        @pl.loop(0, ROWS_PER_W, step=2)
        def _(r):
            pltpu.make_async_copy(tbl_hbm.at[idx_v.at[r]], rows_a, sem_a).wait()
            _accum(rows_a, acc_v, r)

            @pl.when(r + 2 < ROWS_PER_W)
            def _():
                pltpu.async_copy(tbl_hbm.at[idx_v.at[r + 2]], rows_a, sem_a)

            pltpu.make_async_copy(
                tbl_hbm.at[idx_v.at[r + 1]], rows_b, sem_b
            ).wait()
            _accum(rows_b, acc_v, r + 1)

            @pl.when(r + 3 < ROWS_PER_W)
            def _():
                pltpu.async_copy(tbl_hbm.at[idx_v.at[r + 3]], rows_b, sem_b)

        pltpu.sync_copy(acc_v, out_hbm.at[pl.ds(base, ROWS_PER_W)])

    return k(x, emb_table)


VB = 2048
_GRID = (VOCAB + VB - 1) // VB


def _tc_project_t(h, W, brow):
    def mm(h_ref, w_ref, b_ref, o_ref):
        hb = h_ref[...].astype(jnp.bfloat16)
        wb = w_ref[...].astype(jnp.bfloat16)
        acc = lax.dot_general(
            wb, hb, (((1,), (1,)), ((), ())),
            preferred_element_type=jnp.float32,
        )
        o_ref[...] = acc + b_ref[...].T

    return pl.pallas_call(
        mm,
        grid=(_GRID,),
        in_specs=[
            pl.BlockSpec((BATCH, DIM), lambda j: (0, 0)),
            pl.BlockSpec((VB, DIM), lambda j: (j, 0)),
            pl.BlockSpec((1, VB), lambda j: (0, j)),
        ],
        out_specs=pl.BlockSpec((VB, BATCH), lambda j: (j, 0)),
        out_shape=jax.ShapeDtypeStruct((VOCAB, BATCH), jnp.float32),
        compiler_params=pltpu.CompilerParams(
            dimension_semantics=("arbitrary",),
        ),
    )(h, W, brow)


def kernel(x, emb_table, W, b):
    x = x.astype(jnp.int32)
    h = _sc_gather_sum(x, emb_table)
    lt = _tc_project_t(h, W, b.reshape(1, VOCAB))
    return lt.T


# speedup vs baseline: 1.3460x; 1.0166x over previous
import functools

import jax
import jax.numpy as jnp
from jax import lax
from jax.experimental import pallas as pl
from jax.experimental.pallas import tpu as pltpu
from jax.experimental.pallas import tpu_sc as plsc

VOCAB = 100000
DIM = 128
BATCH = 1024
CTX = 50

NC = 2
NS = 16
L = 16
NW = NC * NS
ROWS_PER_W = BATCH // NW


def _sc_gather_sum(x, emb_table):
    mesh = plsc.VectorSubcoreMesh(core_axis_name="c", subcore_axis_name="s")

    def _accum(rows_v, acc_v, r):
        for c in range(DIM // L):
            sl = pl.ds(c * L, L)
            s = rows_v[0, sl]
            for rr in range(1, CTX):
                s = s + rows_v[rr, sl]
            acc_v[r, sl] = s

    @functools.partial(
        pl.kernel,
        out_type=jax.ShapeDtypeStruct((BATCH, DIM), jnp.float32),
        mesh=mesh,
        scratch_types=[
            pltpu.VMEM((ROWS_PER_W, CTX), jnp.int32),
            pltpu.VMEM((CTX, DIM), jnp.float32),
            pltpu.VMEM((CTX, DIM), jnp.float32),
            pltpu.VMEM((ROWS_PER_W, DIM), jnp.float32),
            pltpu.SemaphoreType.DMA,
            pltpu.SemaphoreType.DMA,
        ],
    )
    def k(x_hbm, tbl_hbm, out_hbm, idx_v, rows_a, rows_b, acc_v, sem_a, sem_b):
        wid = lax.axis_index("s") * NC + lax.axis_index("c")
        base = wid * ROWS_PER_W
        pltpu.sync_copy(x_hbm.at[pl.ds(base, ROWS_PER_W)], idx_v)

        pltpu.async_copy(tbl_hbm.at[idx_v.at[0]], rows_a, sem_a)
        pltpu.async_copy(tbl_hbm.at[idx_v.at[1]], rows_b, sem_b)

        @pl.loop(0, ROWS_PER_W, step=2)
        def _(r):
            pltpu.make_async_copy(tbl_hbm.at[idx_v.at[r]], rows_a, sem_a).wait()
            _accum(rows_a, acc_v, r)

            @pl.when(r + 2 < ROWS_PER_W)
            def _():
                pltpu.async_copy(tbl_hbm.at[idx_v.at[r + 2]], rows_a, sem_a)

            pltpu.make_async_copy(
                tbl_hbm.at[idx_v.at[r + 1]], rows_b, sem_b
            ).wait()
            _accum(rows_b, acc_v, r + 1)

            @pl.when(r + 3 < ROWS_PER_W)
            def _():
                pltpu.async_copy(tbl_hbm.at[idx_v.at[r + 3]], rows_b, sem_b)

        pltpu.sync_copy(acc_v, out_hbm.at[pl.ds(base, ROWS_PER_W)])

    return k(x, emb_table)


VB = 4096
_GRID = (VOCAB + VB - 1) // VB


def _tc_project_t(h, W, brow):
    def mm(h_ref, w_ref, b_ref, o_ref):
        hb = h_ref[...].astype(jnp.bfloat16)
        wb = w_ref[...].astype(jnp.bfloat16)
        acc = lax.dot_general(
            wb, hb, (((1,), (1,)), ((), ())),
            preferred_element_type=jnp.float32,
        )
        o_ref[...] = acc + b_ref[...].T

    return pl.pallas_call(
        mm,
        grid=(_GRID,),
        in_specs=[
            pl.BlockSpec((BATCH, DIM), lambda j: (0, 0)),
            pl.BlockSpec((VB, DIM), lambda j: (j, 0)),
            pl.BlockSpec((1, VB), lambda j: (0, j)),
        ],
        out_specs=pl.BlockSpec((VB, BATCH), lambda j: (j, 0)),
        out_shape=jax.ShapeDtypeStruct((VOCAB, BATCH), jnp.float32),
        compiler_params=pltpu.CompilerParams(
            dimension_semantics=("arbitrary",),
        ),
    )(h, W, brow)


def kernel(x, emb_table, W, b):
    x = x.astype(jnp.int32)
    h = _sc_gather_sum(x, emb_table)
    lt = _tc_project_t(h, W, b.reshape(1, VOCAB))
    return lt.T
